# baseline (speedup 1.0000x reference)
import functools

import jax
import jax.numpy as jnp
from jax import lax
from jax.experimental import pallas as pl
from jax.experimental.pallas import tpu as pltpu
from jax.experimental.pallas import tpu_sc as plsc

N_NODES = 10000
STATE = 128
MSG = 64
NODE = 64
MLP = 128
CLIP = 5.0

NC = 2
NS = 16
NW = NC * NS
CHUNK = 128
NPAD = 10112
ZR = NPAD // NS

BLK = 2000


def _pre_body(obs_ref, mean_ref, var_ref, W_ref, b_ref, mW_ref, mb_ref,
              x_ref, m_ref):
    o = obs_ref[...]
    xn = jnp.clip((o - mean_ref[...]) / (jnp.sqrt(var_ref[...]) + 1e-8),
                  -CLIP, CLIP)
    x = jnp.tanh(jnp.dot(xn, W_ref[...], preferred_element_type=jnp.float32)
                 + b_ref[...])
    x_ref[...] = x
    m_ref[...] = jnp.dot(x, mW_ref[...], preferred_element_type=jnp.float32) + mb_ref[...]


def _pre(obs, mean, var, W, b, mW, mb):
    rep = lambda shape: pl.BlockSpec(shape, lambda i: (0, 0))
    return pl.pallas_call(
        _pre_body,
        grid=(N_NODES // BLK,),
        in_specs=[
            pl.BlockSpec((BLK, STATE), lambda i: (i, 0)),
            rep((1, STATE)), rep((1, STATE)),
            rep((STATE, STATE)), rep((1, STATE)),
            rep((STATE, MSG)), rep((1, MSG)),
        ],
        out_specs=[pl.BlockSpec((BLK, STATE), lambda i: (i, 0)),
                   pl.BlockSpec((BLK, MSG), lambda i: (i, 0))],
        out_shape=[jax.ShapeDtypeStruct((N_NODES, STATE), jnp.float32),
                   jax.ShapeDtypeStruct((N_NODES, MSG), jnp.float32)],
    )(obs, mean, var, W, b, mW, mb)


def _node_body(x_ref, p_ref, nWx_ref, nWa_ref, nb_ref, mW_ref, mb_ref,
               x2_ref, m2_ref):
    agg = p_ref[0] + p_ref[1]
    h = jnp.tanh(jnp.dot(x_ref[...], nWx_ref[...], preferred_element_type=jnp.float32)
                 + jnp.dot(agg, nWa_ref[...], preferred_element_type=jnp.float32)
                 + nb_ref[...])
    x2_ref[...] = h
    m2_ref[...] = jnp.dot(h, mW_ref[...], preferred_element_type=jnp.float32) + mb_ref[...]


def _node(x, part, nW, nb, mW, mb):
    d = x.shape[1]
    rep2 = lambda shape: pl.BlockSpec(shape, lambda i: (0, 0))
    return pl.pallas_call(
        _node_body,
        grid=(N_NODES // BLK,),
        in_specs=[
            pl.BlockSpec((BLK, d), lambda i: (i, 0)),
            pl.BlockSpec((NC, BLK, MSG), lambda i: (0, i, 0)),
            rep2((d, NODE)), rep2((MSG, NODE)), rep2((1, NODE)),
            rep2((NODE, MSG)), rep2((1, MSG)),
        ],
        out_specs=[pl.BlockSpec((BLK, NODE), lambda i: (i, 0)),
                   pl.BlockSpec((BLK, MSG), lambda i: (i, 0))],
        out_shape=[jax.ShapeDtypeStruct((N_NODES, NODE), jnp.float32),
                   jax.ShapeDtypeStruct((N_NODES, MSG), jnp.float32)],
    )(x, part, nW[:d], nW[d:], nb, mW, mb)


def _node_last_body(x_ref, p_ref, nWx_ref, nWa_ref, nb_ref, x2_ref):
    agg = p_ref[0] + p_ref[1]
    x2_ref[...] = jnp.tanh(
        jnp.dot(x_ref[...], nWx_ref[...], preferred_element_type=jnp.float32)
        + jnp.dot(agg, nWa_ref[...], preferred_element_type=jnp.float32)
        + nb_ref[...])


def _node_last(x, part, nW, nb):
    d = x.shape[1]
    rep2 = lambda shape: pl.BlockSpec(shape, lambda i: (0, 0))
    return pl.pallas_call(
        _node_last_body,
        grid=(N_NODES // BLK,),
        in_specs=[
            pl.BlockSpec((BLK, d), lambda i: (i, 0)),
            pl.BlockSpec((NC, BLK, MSG), lambda i: (0, i, 0)),
            rep2((d, NODE)), rep2((MSG, NODE)), rep2((1, NODE)),
        ],
        out_specs=pl.BlockSpec((BLK, NODE), lambda i: (i, 0)),
        out_shape=jax.ShapeDtypeStruct((N_NODES, NODE), jnp.float32),
    )(x, part, nW[:d], nW[d:], nb)


def _head_body(x_ref, W_ref, b_ref, vW_ref, vb_ref, out_ref):
    h = jnp.tanh(jnp.dot(x_ref[...], W_ref[...], preferred_element_type=jnp.float32)
                 + b_ref[...])
    out_ref[...] = jnp.dot(h, vW_ref[...], preferred_element_type=jnp.float32) + vb_ref[...]


def _head(xf, W, b, vWp, vb):
    R = xf.shape[0]
    full = lambda shape: pl.BlockSpec(shape, lambda: tuple(0 for _ in shape))
    return pl.pallas_call(
        _head_body,
        in_specs=[full((R, NODE)), full((NODE, MLP)), full((1, MLP)),
                  full((MLP, MLP)), full((1, MLP))],
        out_specs=full((R, MLP)),
        out_shape=jax.ShapeDtypeStruct((R, MLP), jnp.float32),
    )(xf, W, b, vWp, vb)


def _sc_agg(m, pk, cpt0, cpt1):
    mesh = plsc.VectorSubcoreMesh(core_axis_name="c", subcore_axis_name="s")

    NSLOT = 7
    LOOK = 4
    cpt = max(cpt0, cpt1)

    @functools.partial(
        pl.kernel,
        out_type=jax.ShapeDtypeStruct((NC, N_NODES, MSG), jnp.float32),
        mesh=mesh,
        scratch_types=[
            pltpu.VMEM((cpt, CHUNK), jnp.int32),
            [pltpu.VMEM((CHUNK, MSG), jnp.float32) for _ in range(NSLOT)],
            pltpu.VMEM((NSLOT, CHUNK), jnp.int32),
            pltpu.VMEM((NSLOT, CHUNK), jnp.int32),
            pltpu.VMEM_SHARED((NPAD, MSG), jnp.float32),
            [pltpu.SemaphoreType.DMA for _ in range(NSLOT)],
            [pltpu.SemaphoreType.DMA for _ in range(NSLOT)],
        ],
        compiler_params=pltpu.CompilerParams(use_tc_tiling_on_sc=False,
                                             needs_layout_passes=False),
    )
    def k(m_hbm, pk_hbm, out_hbm, pkv, bufs, fvr, tvr, aggsh, gsem, ssem):
        c = lax.axis_index("c")
        s = lax.axis_index("s")

        def zrow(i, _):
            for q in range(MSG // 16):
                bufs[0][i, pl.ds(q * 16, 16)] = jnp.zeros((16,), jnp.float32)
            return 0
        lax.fori_loop(0, CHUNK, zrow, 0)
        for q in range(ZR // CHUNK):
            pltpu.sync_copy(bufs[0], aggsh.at[pl.ds(s * ZR + q * CHUNK, CHUNK)])
        zt = ZR - (ZR // CHUNK) * CHUNK
        if zt:
            pltpu.sync_copy(bufs[0].at[pl.ds(0, zt)],
                            aggsh.at[pl.ds(s * ZR + (ZR // CHUNK) * CHUNK, zt)])
        plsc.subcore_barrier()

        w = c * NS + s
        cptc = jnp.where(c == 0, jnp.int32(cpt0), jnp.int32(cpt1))
        pltpu.sync_copy(pk_hbm.at[w], pkv)

        def unpack(b, j):
            for q in range(CHUNK // 16):
                p = pkv[j, pl.ds(q * 16, 16)]
                fvr[b, pl.ds(q * 16, 16)] = p & jnp.full((16,), 0xFFFF, jnp.int32)
                tvr[b, pl.ds(q * 16, 16)] = lax.shift_right_logical(p, jnp.full((16,), 16, jnp.int32))

        def wait_gather(b):
            pltpu.make_async_copy(m_hbm.at[fvr.at[b]], bufs[b], gsem[b]).wait()

        def wait_scatter(b):
            pltpu.make_async_copy(bufs[b], aggsh.at[tvr.at[b]], ssem[b]).wait()

        for b in range(LOOK):
            unpack(b, jnp.int32(b))
            pltpu.async_copy(m_hbm.at[fvr.at[b]], bufs[b], gsem[b])

        def outer(i, _):
            for b in range(NSLOT):
                kk = i * NSLOT + b

                @pl.when(kk < cptc)
                def _():
                    wait_gather(b)
                    pltpu.make_async_copy(
                        bufs[b], aggsh.at[tvr.at[b]], ssem[b]).start(add=True)
                    k2 = kk + LOOK

                    @pl.when(k2 < cptc)
                    def _():
                        b2 = (b + LOOK) % NSLOT

                        @pl.when(k2 >= NSLOT)
                        def _():
                            wait_scatter(b2)
                        unpack(b2, k2)
                        pltpu.async_copy(m_hbm.at[fvr.at[b2]], bufs[b2], gsem[b2])
            return 0
        lax.fori_loop(0, (cptc + NSLOT - 1) // NSLOT, outer, 0)

        for b in range(NSLOT):
            wait_scatter(b)

        plsc.subcore_barrier()
        row0 = s * ZR

        @pl.when(s < NS - 1)
        def _():
            pltpu.sync_copy(aggsh.at[pl.ds(row0, ZR)],
                            out_hbm.at[c, pl.ds(row0, ZR)])

        @pl.when(s == NS - 1)
        def _():
            tail = ZR - (NPAD - N_NODES)
            pltpu.sync_copy(aggsh.at[pl.ds(row0, tail)],
                            out_hbm.at[c, pl.ds(row0, tail)])

    return k(m, pk)


ROOTS = 512
DUMMY_SLOT = ROOTS - 1


def _sc_agg_root(m, pk, first_hbm, cpt):
    mesh = plsc.VectorSubcoreMesh(core_axis_name="c", subcore_axis_name="s")
    FLAT = cpt * CHUNK
    RPT = ROOTS // NS

    @functools.partial(
        pl.kernel,
        out_type=jax.ShapeDtypeStruct((NC, ROOTS, MSG), jnp.float32),
        mesh=mesh,
        scratch_types=[
            pltpu.VMEM((cpt, CHUNK), jnp.int32),
            pltpu.VMEM((FLAT + 160,), jnp.int32),
            pltpu.VMEM((FLAT + 160,), jnp.int32),
            pltpu.VMEM((cpt, CHUNK), jnp.int32),
            pltpu.VMEM((NPAD,), jnp.int32),
            pltpu.VMEM((ROOTS,), jnp.int32),
            [pltpu.VMEM((CHUNK, MSG), jnp.float32) for _ in range(2)],
            pltpu.VMEM_SHARED((ROOTS, MSG), jnp.float32),
            [pltpu.SemaphoreType.DMA for _ in range(2)],
            [pltpu.SemaphoreType.DMA for _ in range(2)],
        ],
        compiler_params=pltpu.CompilerParams(use_tc_tiling_on_sc=False,
                                             needs_layout_passes=False),
    )
    def k(m_hbm, pk_hbm, fi_hbm, out_hbm, pkv, fc, tc1, tc2, slot,
          fiv, bufs, aggsh, gsem, ssem):
        c = lax.axis_index("c")
        s = lax.axis_index("s")

        def zagg(i, _):
            for q in range(MSG // 16):
                aggv = jnp.zeros((16,), jnp.float32)
                bufs[0][i, pl.ds(q * 16, 16)] = aggv
            return 0
        lax.fori_loop(0, RPT, zagg, 0)
        pltpu.sync_copy(bufs[0].at[pl.ds(0, RPT)], aggsh.at[pl.ds(s * RPT, RPT)])

        def zflag(i, _):
            slot[pl.ds(i * 16, 16)] = jnp.zeros((16,), jnp.int32)
            return 0
        lax.fori_loop(0, NPAD // 16, zflag, 0)
        pltpu.sync_copy(fi_hbm, fiv)
        for q in range(ROOTS // 16):
            idx = fiv[pl.ds(q * 16, 16)]
            val = jax.lax.iota(jnp.int32, 16) + (q * 16 + 1)
            plsc.store_scatter(slot, [idx], val)

        w = c * NS + s
        pltpu.sync_copy(pk_hbm.at[w], pkv)

        def filt_row(j, cnt):
            for q in range(CHUNK // 16):
                p = pkv[j, pl.ds(q * 16, 16)]
                t16 = lax.shift_right_logical(p, jnp.full((16,), 16, jnp.int32))
                f16 = p & jnp.full((16,), 0xFFFF, jnp.int32)
                sl = plsc.load_gather(slot, [t16])
                keep = sl > 0
                plsc.store_compressed(fc.at[pl.ds(cnt, 16)], f16, mask=keep)
                plsc.store_compressed(tc1.at[pl.ds(cnt, 16)], sl - 1, mask=keep)
                cnt = cnt + plsc.all_reduce_population_count(keep)[0]
            return cnt
        cnt = lax.fori_loop(0, cpt, filt_row, jnp.int32(0))

        for q in range(CHUNK // 16 + 1):
            fc[pl.ds(cnt + q * 16, 16)] = jnp.zeros((16,), jnp.int32)
            tc1[pl.ds(cnt + q * 16, 16)] = jnp.full((16,), DUMMY_SLOT, jnp.int32)
        nch = (cnt + CHUNK - 1) // CHUNK

        def trow(r, _):
            for q in range(CHUNK // 16):
                tc2[r, pl.ds(q * 16, 16)] = tc1[pl.ds(r * CHUNK + q * 16, 16)]
            return 0
        lax.fori_loop(0, nch, trow, 0)

        def agg_step(r, _):
            b = 0
            pltpu.async_copy(m_hbm.at[fc.at[pl.ds(r * CHUNK, CHUNK)]],
                             bufs[b], gsem[b])
            pltpu.make_async_copy(m_hbm.at[fc.at[pl.ds(0, CHUNK)]],
                                  bufs[b], gsem[b]).wait()
            pltpu.make_async_copy(bufs[b], aggsh.at[tc2.at[r]],
                                  ssem[b]).start(add=True)
            pltpu.make_async_copy(bufs[b], aggsh.at[tc2.at[0]], ssem[b]).wait()
            return 0
        lax.fori_loop(0, nch, agg_step, 0)

        plsc.subcore_barrier()
        pltpu.sync_copy(aggsh.at[pl.ds(s * RPT, RPT)],
                        out_hbm.at[c, pl.ds(s * RPT, RPT)])

    return k(m, pk, first_hbm)


def _node_head_body(x_ref, p_ref, nWx_ref, nWa_ref, nb_ref, W_ref, b_ref,
                    vW_ref, vb_ref, out_ref):
    agg = p_ref[0] + p_ref[1]
    h = jnp.tanh(jnp.dot(x_ref[...], nWx_ref[...], preferred_element_type=jnp.float32)
                 + jnp.dot(agg, nWa_ref[...], preferred_element_type=jnp.float32)
                 + nb_ref[...])
    h2 = jnp.tanh(jnp.dot(h, W_ref[...], preferred_element_type=jnp.float32)
                  + b_ref[...])
    out_ref[...] = jnp.dot(h2, vW_ref[...], preferred_element_type=jnp.float32) + vb_ref[...]


def _node_head(xf, part, nW, nb, W, b, vWp, vb):
    d = xf.shape[1]
    full = lambda shape: pl.BlockSpec(shape, lambda: tuple(0 for _ in shape))
    return pl.pallas_call(
        _node_head_body,
        in_specs=[full((ROOTS, d)), full((NC, ROOTS, MSG)),
                  full((d, NODE)), full((MSG, NODE)), full((1, NODE)),
                  full((NODE, MLP)), full((1, MLP)),
                  full((MLP, MLP)), full((1, MLP))],
        out_specs=full((ROOTS, MLP)),
        out_shape=jax.ShapeDtypeStruct((ROOTS, MLP), jnp.float32),
    )(xf, part, nW[:d], nW[d:], nb, W, b, vWp, vb)


CPT0 = 230
CPT1 = 84


def _prep_edges(edge_index):
    e = edge_index.shape[1]
    src = edge_index[0]
    dst = edge_index[1]
    fwd = src | (dst << 16)
    bwd = dst | (src << 16)
    grain_total = NS * (CPT0 + CPT1) * CHUNK
    pad = grain_total - 2 * e
    assert pad >= 0
    pk = jnp.concatenate(
        [fwd, bwd, jnp.full((pad,), N_NODES << 16, jnp.int32)])
    cpt = max(CPT0, CPT1)
    n0 = NS * CPT0 * CHUNK
    pk0 = pk[:n0].reshape(NS, CPT0, CHUNK)
    pk1 = pk[n0:].reshape(NS, CPT1, CHUNK)
    dummy = N_NODES << 16
    pk1 = jnp.pad(pk1, ((0, 0), (0, cpt - CPT1), (0, 0)), constant_values=dummy)
    pk0 = jnp.pad(pk0, ((0, 0), (0, cpt - CPT0), (0, 0)), constant_values=dummy)
    cpt_even = (CPT0 + CPT1) // 2
    pk_even = pk.reshape(NW, cpt_even, CHUNK)
    return jnp.concatenate([pk0, pk1], axis=0), cpt, pk_even, cpt_even


def kernel(obs, edge_index, use_transform_action, num_nodes, norm_mean,
           norm_var, pre_W0, pre_b0, g0_mW, g0_mb, g0_nW, g0_nb, g1_mW, g1_mb,
           g1_nW, g1_nb, g2_mW, g2_mb, g2_nW, g2_nb, mlp_W0, mlp_b0, v_W, v_b):
    del use_transform_action
    row = lambda v: v.reshape(1, -1)
    pk, cpt, pk_even, cpt_even = _prep_edges(edge_index)

    num_nodes_cum = jnp.cumsum(num_nodes)
    first_idx = jnp.concatenate(
        [jnp.zeros((1,), num_nodes_cum.dtype), num_nodes_cum[:-1]])
    b = first_idx.shape[0]
    first_pad = jnp.concatenate(
        [first_idx, jnp.full((ROOTS - b,), NPAD - 8, jnp.int32)])

    x, m = _pre(obs, row(norm_mean), row(norm_var), pre_W0, row(pre_b0),
                g0_mW, row(g0_mb))
    part = _sc_agg(m, pk, CPT0, CPT1)
    x, m = _node(x, part, g0_nW, row(g0_nb), g1_mW, row(g1_mb))
    part = _sc_agg(m, pk, CPT0, CPT1)
    x, m = _node(x, part, g1_nW, row(g1_nb), g2_mW, row(g2_mb))
    part = _sc_agg_root(m, pk_even, first_pad, cpt_even)

    xf = jnp.take(x, first_idx, axis=0)
    xf = jnp.pad(xf, ((0, ROOTS - b), (0, 0)))
    vWp = jnp.pad(v_W, ((0, 0), (0, MLP - v_W.shape[1])))
    vbp = jnp.pad(v_b, (0, MLP - v_b.shape[0]), constant_values=v_b[0])
    out = _node_head(xf, part, g2_nW, row(g2_nb), mlp_W0, row(mlp_b0),
                     vWp, row(vbp))
    return out[:b, :1]


# speedup vs baseline: 1.0170x; 1.0170x over previous
import functools

import jax
import jax.numpy as jnp
from jax import lax
from jax.experimental import pallas as pl
from jax.experimental.pallas import tpu as pltpu
from jax.experimental.pallas import tpu_sc as plsc

N_NODES = 10000
STATE = 128
MSG = 64
NODE = 64
MLP = 128
CLIP = 5.0

NC = 2
NS = 16
NW = NC * NS
CHUNK = 128
NPAD = 10112
ZR = NPAD // NS

BLK = 2000


def _pre_body(obs_ref, mean_ref, var_ref, W_ref, b_ref, mW_ref, mb_ref,
              x_ref, m_ref):
    o = obs_ref[...]
    xn = jnp.clip((o - mean_ref[...]) / (jnp.sqrt(var_ref[...]) + 1e-8),
                  -CLIP, CLIP)
    x = jnp.tanh(jnp.dot(xn, W_ref[...], preferred_element_type=jnp.float32)
                 + b_ref[...])
    x_ref[...] = x
    m_ref[...] = jnp.dot(x, mW_ref[...], preferred_element_type=jnp.float32) + mb_ref[...]


def _pre(obs, mean, var, W, b, mW, mb):
    rep = lambda shape: pl.BlockSpec(shape, lambda i: (0, 0))
    return pl.pallas_call(
        _pre_body,
        grid=(N_NODES // BLK,),
        in_specs=[
            pl.BlockSpec((BLK, STATE), lambda i: (i, 0)),
            rep((1, STATE)), rep((1, STATE)),
            rep((STATE, STATE)), rep((1, STATE)),
            rep((STATE, MSG)), rep((1, MSG)),
        ],
        out_specs=[pl.BlockSpec((BLK, STATE), lambda i: (i, 0)),
                   pl.BlockSpec((BLK, MSG), lambda i: (i, 0))],
        out_shape=[jax.ShapeDtypeStruct((N_NODES, STATE), jnp.float32),
                   jax.ShapeDtypeStruct((N_NODES, MSG), jnp.float32)],
    )(obs, mean, var, W, b, mW, mb)


def _node_body(x_ref, p_ref, nWx_ref, nWa_ref, nb_ref, mW_ref, mb_ref,
               x2_ref, m2_ref):
    agg = p_ref[0] + p_ref[1]
    h = jnp.tanh(jnp.dot(x_ref[...], nWx_ref[...], preferred_element_type=jnp.float32)
                 + jnp.dot(agg, nWa_ref[...], preferred_element_type=jnp.float32)
                 + nb_ref[...])
    x2_ref[...] = h
    m2_ref[...] = jnp.dot(h, mW_ref[...], preferred_element_type=jnp.float32) + mb_ref[...]


def _node(x, part, nW, nb, mW, mb):
    d = x.shape[1]
    rep2 = lambda shape: pl.BlockSpec(shape, lambda i: (0, 0))
    return pl.pallas_call(
        _node_body,
        grid=(N_NODES // BLK,),
        in_specs=[
            pl.BlockSpec((BLK, d), lambda i: (i, 0)),
            pl.BlockSpec((NC, BLK, MSG), lambda i: (0, i, 0)),
            rep2((d, NODE)), rep2((MSG, NODE)), rep2((1, NODE)),
            rep2((NODE, MSG)), rep2((1, MSG)),
        ],
        out_specs=[pl.BlockSpec((BLK, NODE), lambda i: (i, 0)),
                   pl.BlockSpec((BLK, MSG), lambda i: (i, 0))],
        out_shape=[jax.ShapeDtypeStruct((N_NODES, NODE), jnp.float32),
                   jax.ShapeDtypeStruct((N_NODES, MSG), jnp.float32)],
    )(x, part, nW[:d], nW[d:], nb, mW, mb)


def _sc_agg(m, pk, cpt0, cpt1):
    mesh = plsc.VectorSubcoreMesh(core_axis_name="c", subcore_axis_name="s")

    NSLOT = 7
    LOOK = 4
    cpt = max(cpt0, cpt1)

    @functools.partial(
        pl.kernel,
        out_type=jax.ShapeDtypeStruct((NC, N_NODES, MSG), jnp.float32),
        mesh=mesh,
        scratch_types=[
            pltpu.VMEM((cpt, CHUNK), jnp.int32),
            [pltpu.VMEM((CHUNK, MSG), jnp.float32) for _ in range(NSLOT)],
            pltpu.VMEM((NSLOT, CHUNK), jnp.int32),
            pltpu.VMEM((NSLOT, CHUNK), jnp.int32),
            pltpu.VMEM_SHARED((NPAD, MSG), jnp.float32),
            [pltpu.SemaphoreType.DMA for _ in range(NSLOT)],
            [pltpu.SemaphoreType.DMA for _ in range(NSLOT)],
        ],
        compiler_params=pltpu.CompilerParams(use_tc_tiling_on_sc=False,
                                             needs_layout_passes=False),
    )
    def k(m_hbm, pk_hbm, out_hbm, pkv, bufs, fvr, tvr, aggsh, gsem, ssem):
        c = lax.axis_index("c")
        s = lax.axis_index("s")

        def zrow(i, _):
            for q in range(MSG // 16):
                bufs[0][i, pl.ds(q * 16, 16)] = jnp.zeros((16,), jnp.float32)
            return 0
        lax.fori_loop(0, CHUNK, zrow, 0)
        for q in range(ZR // CHUNK):
            pltpu.sync_copy(bufs[0], aggsh.at[pl.ds(s * ZR + q * CHUNK, CHUNK)])
        zt = ZR - (ZR // CHUNK) * CHUNK
        if zt:
            pltpu.sync_copy(bufs[0].at[pl.ds(0, zt)],
                            aggsh.at[pl.ds(s * ZR + (ZR // CHUNK) * CHUNK, zt)])
        plsc.subcore_barrier()

        w = c * NS + s
        cptc = jnp.where(c == 0, jnp.int32(cpt0), jnp.int32(cpt1))
        pltpu.sync_copy(pk_hbm.at[w], pkv)

        def unpack(b, j):
            for q in range(CHUNK // 16):
                p = pkv[j, pl.ds(q * 16, 16)]
                fvr[b, pl.ds(q * 16, 16)] = p & jnp.full((16,), 0xFFFF, jnp.int32)
                tvr[b, pl.ds(q * 16, 16)] = lax.shift_right_logical(p, jnp.full((16,), 16, jnp.int32))

        def wait_gather(b):
            pltpu.make_async_copy(m_hbm.at[fvr.at[b]], bufs[b], gsem[b]).wait()

        def wait_scatter(b):
            pltpu.make_async_copy(bufs[b], aggsh.at[tvr.at[b]], ssem[b]).wait()

        for b in range(LOOK):
            unpack(b, jnp.int32(b))
            pltpu.async_copy(m_hbm.at[fvr.at[b]], bufs[b], gsem[b])

        def outer(i, _):
            for b in range(NSLOT):
                kk = i * NSLOT + b

                @pl.when(kk < cptc)
                def _():
                    wait_gather(b)
                    pltpu.make_async_copy(
                        bufs[b], aggsh.at[tvr.at[b]], ssem[b]).start(add=True)
                    k2 = kk + LOOK

                    @pl.when(k2 < cptc)
                    def _():
                        b2 = (b + LOOK) % NSLOT

                        @pl.when(k2 >= NSLOT)
                        def _():
                            wait_scatter(b2)
                        unpack(b2, k2)
                        pltpu.async_copy(m_hbm.at[fvr.at[b2]], bufs[b2], gsem[b2])
            return 0
        lax.fori_loop(0, (cptc + NSLOT - 1) // NSLOT, outer, 0)

        for b in range(NSLOT):
            wait_scatter(b)

        plsc.subcore_barrier()
        row0 = s * ZR

        @pl.when(s < NS - 1)
        def _():
            pltpu.sync_copy(aggsh.at[pl.ds(row0, ZR)],
                            out_hbm.at[c, pl.ds(row0, ZR)])

        @pl.when(s == NS - 1)
        def _():
            tail = ZR - (NPAD - N_NODES)
            pltpu.sync_copy(aggsh.at[pl.ds(row0, tail)],
                            out_hbm.at[c, pl.ds(row0, tail)])

    return k(m, pk)


ROOTS = 512
DUMMY_SLOT = ROOTS - 1


def _sc_agg_root(m, pk, first_hbm, cpt):
    mesh = plsc.VectorSubcoreMesh(core_axis_name="c", subcore_axis_name="s")
    FLAT = cpt * CHUNK
    RPT = ROOTS // NS

    @functools.partial(
        pl.kernel,
        out_type=jax.ShapeDtypeStruct((NC, ROOTS, MSG), jnp.float32),
        mesh=mesh,
        scratch_types=[
            pltpu.VMEM((cpt, CHUNK), jnp.int32),
            pltpu.VMEM((FLAT + 160,), jnp.int32),
            pltpu.VMEM((FLAT + 160,), jnp.int32),
            pltpu.VMEM((cpt, CHUNK), jnp.int32),
            pltpu.VMEM((NPAD,), jnp.int32),
            pltpu.VMEM((ROOTS,), jnp.int32),
            [pltpu.VMEM((CHUNK, MSG), jnp.float32) for _ in range(2)],
            pltpu.VMEM_SHARED((ROOTS, MSG), jnp.float32),
            [pltpu.SemaphoreType.DMA for _ in range(2)],
            [pltpu.SemaphoreType.DMA for _ in range(2)],
        ],
        compiler_params=pltpu.CompilerParams(use_tc_tiling_on_sc=False,
                                             needs_layout_passes=False),
    )
    def k(m_hbm, pk_hbm, fi_hbm, out_hbm, pkv, fc, tc1, tc2, slot,
          fiv, bufs, aggsh, gsem, ssem):
        c = lax.axis_index("c")
        s = lax.axis_index("s")

        def zagg(i, _):
            for q in range(MSG // 16):
                aggv = jnp.zeros((16,), jnp.float32)
                bufs[0][i, pl.ds(q * 16, 16)] = aggv
            return 0
        lax.fori_loop(0, RPT, zagg, 0)
        pltpu.sync_copy(bufs[0].at[pl.ds(0, RPT)], aggsh.at[pl.ds(s * RPT, RPT)])

        def zflag(i, _):
            slot[pl.ds(i * 16, 16)] = jnp.zeros((16,), jnp.int32)
            return 0
        lax.fori_loop(0, NPAD // 16, zflag, 0)
        pltpu.sync_copy(fi_hbm, fiv)
        for q in range(ROOTS // 16):
            idx = fiv[pl.ds(q * 16, 16)]
            val = jax.lax.iota(jnp.int32, 16) + (q * 16 + 1)
            plsc.store_scatter(slot, [idx], val)

        w = c * NS + s
        pltpu.sync_copy(pk_hbm.at[w], pkv)

        def filt_row(j, cnt):
            for q in range(CHUNK // 16):
                p = pkv[j, pl.ds(q * 16, 16)]
                t16 = lax.shift_right_logical(p, jnp.full((16,), 16, jnp.int32))
                f16 = p & jnp.full((16,), 0xFFFF, jnp.int32)
                sl = plsc.load_gather(slot, [t16])
                keep = sl > 0
                plsc.store_compressed(fc.at[pl.ds(cnt, 16)], f16, mask=keep)
                plsc.store_compressed(tc1.at[pl.ds(cnt, 16)], sl - 1, mask=keep)
                cnt = cnt + plsc.all_reduce_population_count(keep)[0]
            return cnt
        cnt = lax.fori_loop(0, cpt, filt_row, jnp.int32(0))

        for q in range(CHUNK // 16 + 1):
            fc[pl.ds(cnt + q * 16, 16)] = jnp.zeros((16,), jnp.int32)
            tc1[pl.ds(cnt + q * 16, 16)] = jnp.full((16,), DUMMY_SLOT, jnp.int32)
        nch = (cnt + CHUNK - 1) // CHUNK

        def trow(r, _):
            for q in range(CHUNK // 16):
                tc2[r, pl.ds(q * 16, 16)] = tc1[pl.ds(r * CHUNK + q * 16, 16)]
            return 0
        lax.fori_loop(0, nch, trow, 0)

        def agg_step(r, _):
            b = 0
            pltpu.async_copy(m_hbm.at[fc.at[pl.ds(r * CHUNK, CHUNK)]],
                             bufs[b], gsem[b])
            pltpu.make_async_copy(m_hbm.at[fc.at[pl.ds(0, CHUNK)]],
                                  bufs[b], gsem[b]).wait()
            pltpu.make_async_copy(bufs[b], aggsh.at[tc2.at[r]],
                                  ssem[b]).start(add=True)
            pltpu.make_async_copy(bufs[b], aggsh.at[tc2.at[0]], ssem[b]).wait()
            return 0
        lax.fori_loop(0, nch, agg_step, 0)

        plsc.subcore_barrier()
        pltpu.sync_copy(aggsh.at[pl.ds(s * RPT, RPT)],
                        out_hbm.at[c, pl.ds(s * RPT, RPT)])

    return k(m, pk, first_hbm)


def _node_head_body(x_ref, p_ref, nWx_ref, nWa_ref, nb_ref, W_ref, b_ref,
                    vW_ref, vb_ref, out_ref):
    agg = p_ref[0] + p_ref[1]
    h = jnp.tanh(jnp.dot(x_ref[...], nWx_ref[...], preferred_element_type=jnp.float32)
                 + jnp.dot(agg, nWa_ref[...], preferred_element_type=jnp.float32)
                 + nb_ref[...])
    h2 = jnp.tanh(jnp.dot(h, W_ref[...], preferred_element_type=jnp.float32)
                  + b_ref[...])
    out_ref[...] = jnp.dot(h2, vW_ref[...], preferred_element_type=jnp.float32) + vb_ref[...]


def _node_head(xf, part, nW, nb, W, b, vWp, vb):
    d = xf.shape[1]
    full = lambda shape: pl.BlockSpec(shape, lambda: tuple(0 for _ in shape))
    return pl.pallas_call(
        _node_head_body,
        in_specs=[full((ROOTS, d)), full((NC, ROOTS, MSG)),
                  full((d, NODE)), full((MSG, NODE)), full((1, NODE)),
                  full((NODE, MLP)), full((1, MLP)),
                  full((MLP, MLP)), full((1, MLP))],
        out_specs=full((ROOTS, MLP)),
        out_shape=jax.ShapeDtypeStruct((ROOTS, MLP), jnp.float32),
    )(xf, part, nW[:d], nW[d:], nb, W, b, vWp, vb)


CPT0 = 224
CPT1 = 90


def _prep_edges(edge_index):
    e = edge_index.shape[1]
    src = edge_index[0]
    dst = edge_index[1]
    fwd = src | (dst << 16)
    bwd = dst | (src << 16)
    grain_total = NS * (CPT0 + CPT1) * CHUNK
    pad = grain_total - 2 * e
    assert pad >= 0
    pk = jnp.concatenate(
        [fwd, bwd, jnp.full((pad,), N_NODES << 16, jnp.int32)])
    cpt = max(CPT0, CPT1)
    n0 = NS * CPT0 * CHUNK
    pk0 = pk[:n0].reshape(NS, CPT0, CHUNK)
    pk1 = pk[n0:].reshape(NS, CPT1, CHUNK)
    dummy = N_NODES << 16
    pk1 = jnp.pad(pk1, ((0, 0), (0, cpt - CPT1), (0, 0)), constant_values=dummy)
    pk0 = jnp.pad(pk0, ((0, 0), (0, cpt - CPT0), (0, 0)), constant_values=dummy)
    cpt_even = (CPT0 + CPT1) // 2
    pk_even = pk.reshape(NW, cpt_even, CHUNK)
    return jnp.concatenate([pk0, pk1], axis=0), cpt, pk_even, cpt_even


def kernel(obs, edge_index, use_transform_action, num_nodes, norm_mean,
           norm_var, pre_W0, pre_b0, g0_mW, g0_mb, g0_nW, g0_nb, g1_mW, g1_mb,
           g1_nW, g1_nb, g2_mW, g2_mb, g2_nW, g2_nb, mlp_W0, mlp_b0, v_W, v_b):
    del use_transform_action
    row = lambda v: v.reshape(1, -1)
    pk, cpt, pk_even, cpt_even = _prep_edges(edge_index)

    num_nodes_cum = jnp.cumsum(num_nodes)
    first_idx = jnp.concatenate(
        [jnp.zeros((1,), num_nodes_cum.dtype), num_nodes_cum[:-1]])
    b = first_idx.shape[0]
    first_pad = jnp.concatenate(
        [first_idx, jnp.full((ROOTS - b,), NPAD - 8, jnp.int32)])

    x, m = _pre(obs, row(norm_mean), row(norm_var), pre_W0, row(pre_b0),
                g0_mW, row(g0_mb))
    part = _sc_agg(m, pk, CPT0, CPT1)
    x, m = _node(x, part, g0_nW, row(g0_nb), g1_mW, row(g1_mb))
    part = _sc_agg(m, pk, CPT0, CPT1)
    x, m = _node(x, part, g1_nW, row(g1_nb), g2_mW, row(g2_mb))
    part = _sc_agg_root(m, pk_even, first_pad, cpt_even)

    xf = jnp.take(x, first_idx, axis=0)
    xf = jnp.pad(xf, ((0, ROOTS - b), (0, 0)))
    vWp = jnp.pad(v_W, ((0, 0), (0, MLP - v_W.shape[1])))
    vbp = jnp.pad(v_b, (0, MLP - v_b.shape[0]), constant_values=v_b[0])
    out = _node_head(xf, part, g2_nW, row(g2_nb), mlp_W0, row(mlp_b0),
                     vWp, row(vbp))
    return out[:b, :1]
